# baseline (speedup 1.0000x reference)
import jax
import jax.numpy as jnp
from jax.experimental import pallas as pl
from jax.experimental.pallas import tpu as pltpu

_NC = 21
_CC = 3
_BS = 8


def _body(x_hbm, o_hbm, in_v, out_v, sem_c, sem_i, sem_o):
    def conf_copy(j):
        return pltpu.make_async_copy(
            x_hbm.at[pl.ds(7 * j, 7)], o_hbm.at[pl.ds(7 * j, 7)], sem_c.at[j])

    def in_copy(k):
        return pltpu.make_async_copy(
            x_hbm.at[pl.ds(_NC, 12), pl.ds(_BS * k, _BS)],
            in_v.at[k], sem_i.at[k])

    def out_copy(k):
        return pltpu.make_async_copy(
            out_v.at[k],
            o_hbm.at[pl.ds(_NC, 4), pl.ds(_BS * k, _BS)], sem_o.at[k])

    for j in range(_CC):
        conf_copy(j).start()
    in_copy(0).start()
    in_copy(1).start()
    for k in range(2):
        in_copy(k).wait()
        x = in_v[k]
        c = x[0:2] * x[6:8] * x[8:10] + x[4:6]
        wh = jnp.exp(x[2:4] * x[10:12]) * x[6:8]
        cs = c * 512.0
        hs = wh * 256.0
        out_v[k, 0:2] = cs - hs
        out_v[k, 2:4] = cs + hs
        out_copy(k).start()
    out_copy(0).wait()
    out_copy(1).wait()
    for j in range(_CC):
        conf_copy(j).wait()


def kernel(prediction):
    b, n, cin = prediction.shape
    xt = prediction.transpose(2, 0, 1)
    outt = pl.pallas_call(
        _body,
        in_specs=[pl.BlockSpec(memory_space=pl.ANY)],
        out_specs=pl.BlockSpec(memory_space=pl.ANY),
        out_shape=jax.ShapeDtypeStruct((_NC + 4, b, n), jnp.float32),
        scratch_shapes=[
            pltpu.VMEM((2, 12, _BS, n), jnp.float32),
            pltpu.VMEM((2, 4, _BS, n), jnp.float32),
            pltpu.SemaphoreType.DMA((_CC,)),
            pltpu.SemaphoreType.DMA((2,)),
            pltpu.SemaphoreType.DMA((2,)),
        ],
    )(xt)
    return outt.transpose(1, 2, 0)


# speedup vs baseline: 33.7163x; 33.7163x over previous
import jax
import jax.numpy as jnp
from jax.experimental import pallas as pl

_NC = 21
_L = 5120


def _decode_tile(x_ref, o_ref):
    x = x_ref[...]
    o_ref[0:_NC] = x[0:_NC]
    dxy = x[21:23]
    dwh = x[23:25]
    axy = x[25:27]
    awh = x[27:29]
    vxy = x[29:31]
    vwh = x[31:33]
    c = dxy * awh * vxy + axy
    wh = jnp.exp(dwh * vwh) * awh
    cs = c * 512.0
    hs = wh * 256.0
    o_ref[21:23] = cs - hs
    o_ref[23:25] = cs + hs


def kernel(prediction):
    b, n, cin = prediction.shape
    xt = prediction.transpose(2, 0, 1)
    outt = pl.pallas_call(
        _decode_tile,
        grid=(pl.cdiv(n, _L),),
        in_specs=[pl.BlockSpec((cin, b, _L), lambda j: (0, 0, j))],
        out_specs=pl.BlockSpec((_NC + 4, b, _L), lambda j: (0, 0, j)),
        out_shape=jax.ShapeDtypeStruct((_NC + 4, b, n), jnp.float32),
    )(xt)
    return outt.transpose(1, 2, 0)
